# baseline (speedup 1.0000x reference)
import functools

import jax
import jax.numpy as jnp
from jax import lax
from jax.experimental import pallas as pl
from jax.experimental.pallas import tpu as pltpu
from jax.experimental.pallas import tpu_sc as plsc

VOCAB = 1000
D = 1000
BATCH = 4096
SEQ = 20

NC, NS = 2, 16
NW = NC * NS
BL = 128
DT = D // 8
GD = 40
NG = D // GD
GT = GD // 8


def _body(idx_hbm, tabT_hbm, out_hbm, idx_v, idxT_v, slab_v, buf0, buf1,
          slab_sem, s0, s1):
    wid = lax.axis_index("s") * NC + lax.axis_index("c")
    pltpu.sync_copy(idx_hbm.at[pl.ds(wid * BL, BL)], idx_v)

    lanes = lax.iota(jnp.int32, 16)
    for s in range(SEQ):
        svec = jnp.full((16,), s, jnp.int32)
        for bv in range(BL // 16):
            rows = lanes + bv * 16
            idxT_v[s, pl.ds(bv * 16, 16)] = plsc.load_gather(idx_v, [rows, svec])

    bufs = (buf0, buf1)
    sems = (s0, s1)

    def store(buf, sem, g, s):
        return pltpu.make_async_copy(
            buf, out_hbm.at[s, pl.ds(g * GT, GT), wid], sem)

    def fill(buf, g, s):
        for bv in range(BL // 16):
            idx16 = idxT_v[s, pl.ds(bv * 16, 16)]
            for dl in range(GD):
                dvec = jnp.full((16,), dl, jnp.int32)
                buf[dl // 8, dl % 8, pl.ds(bv * 16, 16)] = (
                    plsc.load_gather(slab_v, [dvec, idx16]))

    def pair(i, carry):
        g = i // (SEQ // 2)
        sp = i % (SEQ // 2)

        @pl.when(sp == 0)
        def _():
            pltpu.sync_copy(tabT_hbm.at[pl.ds(g * GD, GD)], slab_v)

        for b in range(2):
            s = 2 * sp + b

            @pl.when(i > 0)
            def _():
                store(bufs[b], sems[b], g, s).wait()

            fill(bufs[b], g, s)
            store(bufs[b], sems[b], g, s).start()
        return carry

    n_pairs = NG * (SEQ // 2)
    lax.fori_loop(0, n_pairs, pair, 0)
    store(buf0, s0, NG - 1, SEQ - 2).wait()
    store(buf1, s1, NG - 1, SEQ - 1).wait()


@functools.partial(jax.jit, static_argnums=())
def _gather_rows(idx, tabT):
    k = pl.kernel(
        _body,
        out_type=jax.ShapeDtypeStruct((SEQ, DT, NW, 8, BL), jnp.float32),
        mesh=plsc.VectorSubcoreMesh(core_axis_name="c", subcore_axis_name="s"),
        scratch_types=[
            pltpu.VMEM((BL, SEQ), jnp.int32),
            pltpu.VMEM((SEQ, BL), jnp.int32),
            pltpu.VMEM((GD, VOCAB), jnp.float32),
            pltpu.VMEM((GT, 8, BL), jnp.float32),
            pltpu.VMEM((GT, 8, BL), jnp.float32),
            pltpu.SemaphoreType.DMA,
            pltpu.SemaphoreType.DMA,
            pltpu.SemaphoreType.DMA,
        ],
        compiler_params=pltpu.CompilerParams(
            use_tc_tiling_on_sc=False, needs_layout_passes=False),
    )
    return k(idx, tabT)


def kernel(inputs, embedding_table):
    out5d = _gather_rows(inputs, embedding_table.T)
    return out5d.transpose(2, 4, 0, 1, 3).reshape(BATCH, SEQ, D)


# speedup vs baseline: 1.3278x; 1.3278x over previous
import functools

import jax
import jax.numpy as jnp
from jax import lax
from jax.experimental import pallas as pl
from jax.experimental.pallas import tpu as pltpu
from jax.experimental.pallas import tpu_sc as plsc

VOCAB = 1000
D = 1000
BATCH = 4096
SEQ = 20

NC, NS = 2, 16
NW = NC * NS
BL = 128
DT = D // 8
GD = 40
NG = D // GD
GT = GD // 8


def _body(idx_hbm, tabT_hbm, out_hbm, idx_v, idxT_v, slab_v, buf0, buf1,
          slab_sem, s0, s1):
    wid = lax.axis_index("s") * NC + lax.axis_index("c")
    pltpu.sync_copy(idx_hbm.at[pl.ds(wid * BL, BL)], idx_v)

    lanes = lax.iota(jnp.int32, 16)
    for s in range(SEQ):
        svec = jnp.full((16,), s, jnp.int32)
        for bv in range(BL // 16):
            rows = lanes + bv * 16
            idxT_v[s, pl.ds(bv * 16, 16)] = plsc.load_gather(idx_v, [rows, svec])

    bufs = (buf0, buf1)
    sems = (s0, s1)

    def store(buf, sem, g, s):
        return pltpu.make_async_copy(
            buf, out_hbm.at[s, pl.ds(g * GT, GT), wid], sem)

    def fill(buf, g, s):
        for bv in range(BL // 16):
            idx16 = idxT_v[s, pl.ds(bv * 16, 16)]
            for dl in range(GD):
                buf[dl // 8, dl % 8, pl.ds(bv * 16, 16)] = (
                    plsc.load_gather(slab_v.at[dl], [idx16]))

    def pair(i, carry):
        g = i // (SEQ // 2)
        sp = i % (SEQ // 2)

        @pl.when(sp == 0)
        def _():
            pltpu.sync_copy(tabT_hbm.at[pl.ds(g * GD, GD)], slab_v)

        for b in range(2):
            s = 2 * sp + b

            @pl.when(i > 0)
            def _():
                store(bufs[b], sems[b], g, s).wait()

            fill(bufs[b], g, s)
            store(bufs[b], sems[b], g, s).start()
        return carry

    n_pairs = NG * (SEQ // 2)
    lax.fori_loop(0, n_pairs, pair, 0)
    store(buf0, s0, NG - 1, SEQ - 2).wait()
    store(buf1, s1, NG - 1, SEQ - 1).wait()


@functools.partial(jax.jit, static_argnums=())
def _gather_rows(idx, tabT):
    k = pl.kernel(
        _body,
        out_type=jax.ShapeDtypeStruct((SEQ, DT, NW, 8, BL), jnp.float32),
        mesh=plsc.VectorSubcoreMesh(core_axis_name="c", subcore_axis_name="s"),
        scratch_types=[
            pltpu.VMEM((BL, SEQ), jnp.int32),
            pltpu.VMEM((SEQ, BL), jnp.int32),
            pltpu.VMEM((GD, VOCAB), jnp.float32),
            pltpu.VMEM((GT, 8, BL), jnp.float32),
            pltpu.VMEM((GT, 8, BL), jnp.float32),
            pltpu.SemaphoreType.DMA,
            pltpu.SemaphoreType.DMA,
            pltpu.SemaphoreType.DMA,
        ],
        compiler_params=pltpu.CompilerParams(
            use_tc_tiling_on_sc=False, needs_layout_passes=False),
    )
    return k(idx, tabT)


def kernel(inputs, embedding_table):
    out5d = _gather_rows(inputs, embedding_table.T)
    return out5d.transpose(2, 4, 0, 1, 3).reshape(BATCH, SEQ, D)


# speedup vs baseline: 3.1065x; 2.3395x over previous
import functools

import jax
import jax.numpy as jnp
from jax import lax
from jax.experimental import pallas as pl
from jax.experimental.pallas import tpu as pltpu
from jax.experimental.pallas import tpu_sc as plsc

VOCAB = 1000
D = 1000
BATCH = 4096
SEQ = 20

NC, NS = 2, 16
NW = NC * NS
BL = 128
DT = D // 8
GD = 40
NG = D // GD
GT = GD // 8


def _body(idx_hbm, tabT_hbm, out_hbm, idx_v, idxT_v, slab_v, buf0, buf1,
          slab_sem, s0, s1):
    wid = lax.axis_index("s") * NC + lax.axis_index("c")
    pltpu.sync_copy(idx_hbm.at[pl.ds(wid * BL, BL)], idx_v)

    lanes = lax.iota(jnp.int32, 16)
    for s in range(SEQ):
        svec = jnp.full((16,), s, jnp.int32)
        for bv in range(BL // 16):
            rows = lanes + bv * 16
            idxT_v[s, pl.ds(bv * 16, 16)] = plsc.load_gather(idx_v, [rows, svec])

    bufs = (buf0, buf1)
    sems = (s0, s1)

    def store(buf, sem, g, s):
        return pltpu.make_async_copy(
            buf, out_hbm.at[s, pl.ds(g * GT, GT), wid], sem)

    def fill(buf, g, s):
        for bv in range(BL // 16):
            idx16 = idxT_v[s, pl.ds(bv * 16, 16)]
            vals = [plsc.load_gather(slab_v.at[dl], [idx16])
                    for dl in range(GD)]
            for dl in range(GD):
                buf[dl // 8, dl % 8, pl.ds(bv * 16, 16)] = vals[dl]

    def pair(i, carry):
        g = i // (SEQ // 2)
        sp = i % (SEQ // 2)

        @pl.when(sp == 0)
        def _():
            pltpu.sync_copy(tabT_hbm.at[pl.ds(g * GD, GD)], slab_v)

        for b in range(2):
            s = 2 * sp + b

            @pl.when(i > 0)
            def _():
                store(bufs[b], sems[b], g, s).wait()

            fill(bufs[b], g, s)
            store(bufs[b], sems[b], g, s).start()
        return carry

    n_pairs = NG * (SEQ // 2)
    lax.fori_loop(0, n_pairs, pair, 0)
    store(buf0, s0, NG - 1, SEQ - 2).wait()
    store(buf1, s1, NG - 1, SEQ - 1).wait()


@functools.partial(jax.jit, static_argnums=())
def _gather_rows(idx, tabT):
    k = pl.kernel(
        _body,
        out_type=jax.ShapeDtypeStruct((SEQ, DT, NW, 8, BL), jnp.float32),
        mesh=plsc.VectorSubcoreMesh(core_axis_name="c", subcore_axis_name="s"),
        scratch_types=[
            pltpu.VMEM((BL, SEQ), jnp.int32),
            pltpu.VMEM((SEQ, BL), jnp.int32),
            pltpu.VMEM((GD, VOCAB), jnp.float32),
            pltpu.VMEM((GT, 8, BL), jnp.float32),
            pltpu.VMEM((GT, 8, BL), jnp.float32),
            pltpu.SemaphoreType.DMA,
            pltpu.SemaphoreType.DMA,
            pltpu.SemaphoreType.DMA,
        ],
        compiler_params=pltpu.CompilerParams(
            use_tc_tiling_on_sc=False, needs_layout_passes=False),
    )
    return k(idx, tabT)


def kernel(inputs, embedding_table):
    out5d = _gather_rows(inputs, embedding_table.T)
    return out5d.transpose(2, 4, 0, 1, 3).reshape(BATCH, SEQ, D)


# speedup vs baseline: 3.6377x; 1.1710x over previous
import functools

import jax
import jax.numpy as jnp
from jax import lax
from jax.experimental import pallas as pl
from jax.experimental.pallas import tpu as pltpu
from jax.experimental.pallas import tpu_sc as plsc

VOCAB = 1000
D = 1000
BATCH = 4096
SEQ = 20

NC, NS = 2, 16
NW = NC * NS
BL = 128
DT = D // 8
GD = 40
NG = D // GD
GT = GD // 8


def _body(idx_hbm, tabT_hbm, out_hbm, idx_v, idxT_v, slab_v, buf0, buf1,
          slab_sem, s0, s1):
    wid = lax.axis_index("s") * NC + lax.axis_index("c")
    pltpu.sync_copy(idx_hbm.at[pl.ds(wid * BL, BL)], idx_v)

    lanes = lax.iota(jnp.int32, 16)
    for s in range(SEQ):
        svec = jnp.full((16,), s, jnp.int32)
        for bv in range(BL // 16):
            rows = lanes + bv * 16
            idxT_v[s, pl.ds(bv * 16, 16)] = plsc.load_gather(idx_v, [rows, svec])

    bufs = (buf0, buf1)
    sems = (s0, s1)

    def store(buf, sem, g, s):
        return pltpu.make_async_copy(
            buf, out_hbm.at[s, pl.ds(g * GT, GT), wid], sem)

    def fill(buf, g, s):
        pend = None
        for bv in range(BL // 16):
            idx16 = idxT_v[s, pl.ds(bv * 16, 16)]
            vals = []
            for dl in range(GD):
                vals.append(plsc.load_gather(slab_v.at[dl], [idx16]))
                if pend is not None:
                    buf[dl // 8, dl % 8, pl.ds(pend[0] * 16, 16)] = pend[1][dl]
            pend = (bv, vals)
        for dl in range(GD):
            buf[dl // 8, dl % 8, pl.ds(pend[0] * 16, 16)] = pend[1][dl]

    def pair(i, carry):
        g = i // (SEQ // 2)
        sp = i % (SEQ // 2)

        @pl.when(sp == 0)
        def _():
            pltpu.sync_copy(tabT_hbm.at[pl.ds(g * GD, GD)], slab_v)

        for b in range(2):
            s = 2 * sp + b

            @pl.when(i > 0)
            def _():
                store(bufs[b], sems[b], g, s).wait()

            fill(bufs[b], g, s)
            store(bufs[b], sems[b], g, s).start()
        return carry

    n_pairs = NG * (SEQ // 2)
    lax.fori_loop(0, n_pairs, pair, 0)
    store(buf0, s0, NG - 1, SEQ - 2).wait()
    store(buf1, s1, NG - 1, SEQ - 1).wait()


@functools.partial(jax.jit, static_argnums=())
def _gather_rows(idx, tabT):
    k = pl.kernel(
        _body,
        out_type=jax.ShapeDtypeStruct((SEQ, DT, NW, 8, BL), jnp.float32),
        mesh=plsc.VectorSubcoreMesh(core_axis_name="c", subcore_axis_name="s"),
        scratch_types=[
            pltpu.VMEM((BL, SEQ), jnp.int32),
            pltpu.VMEM((SEQ, BL), jnp.int32),
            pltpu.VMEM((GD, VOCAB), jnp.float32),
            pltpu.VMEM((GT, 8, BL), jnp.float32),
            pltpu.VMEM((GT, 8, BL), jnp.float32),
            pltpu.SemaphoreType.DMA,
            pltpu.SemaphoreType.DMA,
            pltpu.SemaphoreType.DMA,
        ],
        compiler_params=pltpu.CompilerParams(
            use_tc_tiling_on_sc=False, needs_layout_passes=False),
    )
    return k(idx, tabT)


def kernel(inputs, embedding_table):
    out5d = _gather_rows(inputs, embedding_table.T)
    return out5d.transpose(2, 4, 0, 1, 3).reshape(BATCH, SEQ, D)


# speedup vs baseline: 4.3900x; 1.2068x over previous
import functools

import jax
import jax.numpy as jnp
from jax import lax
from jax.experimental import pallas as pl
from jax.experimental.pallas import tpu as pltpu
from jax.experimental.pallas import tpu_sc as plsc

VOCAB = 1000
D = 1000
BATCH = 4096
SEQ = 20

NC, NS = 2, 16
NW = NC * NS
BL = 128
DT = D // 8
GD = 40
NG = D // GD
GT = GD // 8


def _body(idx_hbm, tabT_hbm, out_hbm, idx_v, idxT_v, slab0, slab1,
          buf0, buf1, ga, gb, s0, s1):
    wid = lax.axis_index("s") * NC + lax.axis_index("c")
    pltpu.sync_copy(idx_hbm.at[pl.ds(wid * BL, BL)], idx_v)

    lanes = lax.iota(jnp.int32, 16)
    for s in range(SEQ):
        svec = jnp.full((16,), s, jnp.int32)
        for bv in range(BL // 16):
            rows = lanes + bv * 16
            idxT_v[s, pl.ds(bv * 16, 16)] = plsc.load_gather(idx_v, [rows, svec])

    bufs = (buf0, buf1)
    sems = (s0, s1)

    def slab_load(g, slab, sem):
        return pltpu.make_async_copy(tabT_hbm.at[pl.ds(g * GD, GD)], slab, sem)

    def store(buf, sem, g, s):
        return pltpu.make_async_copy(
            buf, out_hbm.at[s, pl.ds(g * GT, GT), wid], sem)

    def fill(buf, slab, s):
        pend = None
        for bv in range(BL // 16):
            idx16 = idxT_v[s, pl.ds(bv * 16, 16)]
            vals = []
            for dl in range(GD):
                vals.append(plsc.load_gather(slab.at[dl], [idx16]))
                if pend is not None:
                    buf[dl // 8, dl % 8, pl.ds(pend[0] * 16, 16)] = pend[1][dl]
            pend = (bv, vals)
        for dl in range(GD):
            buf[dl // 8, dl % 8, pl.ds(pend[0] * 16, 16)] = pend[1][dl]

    def run_g(slab, g):
        def pair(sp, carry):
            for b in range(2):
                s = 2 * sp + b

                @pl.when((g > 0) | (sp > 0))
                def _():
                    store(bufs[b], sems[b], g, s).wait()

                fill(bufs[b], slab, s)
                store(bufs[b], sems[b], g, s).start()
            return carry

        lax.fori_loop(0, SEQ // 2, pair, 0)

    slab_load(0, slab0, ga).start()

    def gpair(j, carry):
        g0 = 2 * j
        slab_load(g0, slab0, ga).wait()
        slab_load(g0 + 1, slab1, gb).start()
        run_g(slab0, g0)
        slab_load(g0 + 1, slab1, gb).wait()
        slab_load(g0 + 2, slab0, ga).start()
        run_g(slab1, g0 + 1)
        return carry

    lax.fori_loop(0, (NG - 1) // 2, gpair, 0)
    slab_load(NG - 1, slab0, ga).wait()
    run_g(slab0, NG - 1)
    store(buf0, s0, NG - 1, SEQ - 2).wait()
    store(buf1, s1, NG - 1, SEQ - 1).wait()


@functools.partial(jax.jit, static_argnums=())
def _gather_rows(idx, tabT):
    k = pl.kernel(
        _body,
        out_type=jax.ShapeDtypeStruct((SEQ, DT, NW, 8, BL), jnp.float32),
        mesh=plsc.VectorSubcoreMesh(core_axis_name="c", subcore_axis_name="s"),
        scratch_types=[
            pltpu.VMEM((BL, SEQ), jnp.int32),
            pltpu.VMEM((SEQ, BL), jnp.int32),
            pltpu.VMEM((GD, VOCAB), jnp.float32),
            pltpu.VMEM((GD, VOCAB), jnp.float32),
            pltpu.VMEM((GT, 8, BL), jnp.float32),
            pltpu.VMEM((GT, 8, BL), jnp.float32),
            pltpu.SemaphoreType.DMA,
            pltpu.SemaphoreType.DMA,
            pltpu.SemaphoreType.DMA,
            pltpu.SemaphoreType.DMA,
        ],
        compiler_params=pltpu.CompilerParams(
            use_tc_tiling_on_sc=False, needs_layout_passes=False),
    )
    return k(idx, tabT)


def kernel(inputs, embedding_table):
    out5d = _gather_rows(inputs, embedding_table.T)
    return out5d.transpose(2, 4, 0, 1, 3).reshape(BATCH, SEQ, D)
